# Initial kernel scaffold; baseline (speedup 1.0000x reference)
import jax
import jax.numpy as jnp
from jax.experimental import pallas as pl


def kernel(x, edge_index, batch, W1_1, b1_1, W2_1, b2_1, g_1, bt_1, W1_2, b1_2, W2_2, b2_2, g_2, bt_2, W1_3, b1_3, W2_3, b2_3, g_3, bt_3, W1_4, b1_4, W2_4, b2_4, g_4, bt_4, W1_5, b1_5, W2_5, b2_5, g_5, bt_5, fc1_W, fc1_b, fc2_W, fc2_b):
    raise NotImplementedError("write your pallas kernel here")


# speedup vs baseline: 1.0843x; 1.0843x over previous
import functools

import jax
import jax.numpy as jnp
from jax.experimental import pallas as pl
from jax.experimental.pallas import tpu as pltpu

N_NODES = 10000
DIM = 32
NGRAPH = 64
NCLS = 10


def _proj_body(x_ref, w_ref, o_ref):
    o_ref[...] = jnp.dot(x_ref[...], w_ref[...],
                         preferred_element_type=jnp.float32)


def _proj(x, w):
    return pl.pallas_call(
        _proj_body,
        out_shape=jax.ShapeDtypeStruct((x.shape[0], w.shape[1]), jnp.float32),
    )(x, w)


def _layer_body(first, inp_ref, agg_ref, w1_ref, b1_ref, w2_ref, b2_ref,
                g_ref, bt_ref, o_ref):
    u = inp_ref[...] + agg_ref[0] + agg_ref[1]
    if first:
        h2 = jnp.maximum(u + b1_ref[...], 0.0)
    else:
        h2 = jnp.maximum(
            jnp.dot(u, w1_ref[...], preferred_element_type=jnp.float32)
            + b1_ref[...], 0.0)
    h2 = jnp.dot(h2, w2_ref[...], preferred_element_type=jnp.float32) \
        + b2_ref[...]
    h2 = jnp.maximum(h2, 0.0)
    mu = jnp.mean(h2, axis=0, keepdims=True)
    var = jnp.mean((h2 - mu) ** 2, axis=0, keepdims=True)
    o_ref[...] = g_ref[...] * (h2 - mu) / jnp.sqrt(var + 1e-5) + bt_ref[...]


def _layer(first, inp, agg2, w1, b1, w2, b2, g, bt):
    return pl.pallas_call(
        functools.partial(_layer_body, first),
        out_shape=jax.ShapeDtypeStruct((N_NODES, DIM), jnp.float32),
    )(inp, agg2, w1, b1.reshape(1, DIM), w2, b2.reshape(1, DIM),
      g.reshape(1, DIM), bt.reshape(1, DIM))


def _head_body(h_ref, batch_ref, fc1w_ref, fc1b_ref, fc2w_ref, fc2b_ref,
               o_ref):
    h = h_ref[...]
    batch = batch_ref[...]
    gids = jax.lax.broadcasted_iota(jnp.int32, (NGRAPH, N_NODES), 0)
    onehot = jnp.where(gids == batch, 1.0, 0.0).astype(jnp.float32)
    pooled_sum = jnp.dot(onehot, h, preferred_element_type=jnp.float32)
    cnt = jnp.sum(onehot, axis=1, keepdims=True)
    pooled = pooled_sum / jnp.maximum(cnt, 1.0)
    z = jnp.maximum(
        jnp.dot(pooled, fc1w_ref[...], preferred_element_type=jnp.float32)
        + fc1b_ref[...], 0.0)
    z = jnp.dot(z, fc2w_ref[...], preferred_element_type=jnp.float32) \
        + fc2b_ref[...]
    m = jnp.max(z, axis=-1, keepdims=True)
    lse = jnp.log(jnp.sum(jnp.exp(z - m), axis=-1, keepdims=True)) + m
    o_ref[...] = z - lse


def _head(h, batch, fc1w, fc1b, fc2w, fc2b):
    return pl.pallas_call(
        _head_body,
        out_shape=jax.ShapeDtypeStruct((NGRAPH, NCLS), jnp.float32),
    )(h, batch.reshape(1, N_NODES), fc1w, fc1b.reshape(1, DIM), fc2w,
      fc2b.reshape(1, NCLS))


def _segsum(h, src, dst):
    agg = jax.ops.segment_sum(h[src], dst, num_segments=N_NODES)
    return jnp.stack([agg, jnp.zeros_like(agg)])


def kernel(x, edge_index, batch,
           W1_1, b1_1, W2_1, b2_1, g_1, bt_1,
           W1_2, b1_2, W2_2, b2_2, g_2, bt_2,
           W1_3, b1_3, W2_3, b2_3, g_3, bt_3,
           W1_4, b1_4, W2_4, b2_4, g_4, bt_4,
           W1_5, b1_5, W2_5, b2_5, g_5, bt_5,
           fc1_W, fc1_b, fc2_W, fc2_b):
    src = edge_index[0]
    dst = edge_index[1]
    params = [
        (W1_1, b1_1, W2_1, b2_1, g_1, bt_1),
        (W1_2, b1_2, W2_2, b2_2, g_2, bt_2),
        (W1_3, b1_3, W2_3, b2_3, g_3, bt_3),
        (W1_4, b1_4, W2_4, b2_4, g_4, bt_4),
        (W1_5, b1_5, W2_5, b2_5, g_5, bt_5),
    ]
    h = _proj(x, W1_1)
    for l, (w1, b1, w2, b2, g, bt) in enumerate(params):
        agg2 = _segsum(h, src, dst)
        h = _layer(l == 0, h, agg2, w1, b1, w2, b2, g, bt)
    return _head(h, batch, fc1_W, fc1_b, fc2_W, fc2_b)


# speedup vs baseline: 17.6373x; 16.2660x over previous
import functools

import jax
import jax.numpy as jnp
from jax import lax
from jax.experimental import pallas as pl
from jax.experimental.pallas import tpu as pltpu
from jax.experimental.pallas import tpu_sc as plsc

N_NODES = 10000
DIM = 32
NGRAPH = 64
NCLS = 10
N_EDGES = 320000

_NC = 2
_NS = 16
_NW = _NC * _NS
_EPW = N_EDGES // _NW
_CH = 2000
_NCHUNK = _EPW // _CH
_NPAD = 10240
_ZROWS = _NPAD // _NS


def _proj_body(x_ref, w_ref, o_ref):
    o_ref[...] = jnp.dot(x_ref[...], w_ref[...],
                         preferred_element_type=jnp.float32)


def _proj(x, w):
    return pl.pallas_call(
        _proj_body,
        out_shape=jax.ShapeDtypeStruct((x.shape[0], w.shape[1]), jnp.float32),
    )(x, w)


def _layer_body(first, inp_ref, agg_ref, w1_ref, b1_ref, w2_ref, b2_ref,
                g_ref, bt_ref, o_ref):
    u = inp_ref[...] + agg_ref[0, :N_NODES] + agg_ref[1, :N_NODES]
    if first:
        h2 = jnp.maximum(u + b1_ref[...], 0.0)
    else:
        h2 = jnp.maximum(
            jnp.dot(u, w1_ref[...], preferred_element_type=jnp.float32)
            + b1_ref[...], 0.0)
    h2 = jnp.dot(h2, w2_ref[...], preferred_element_type=jnp.float32) \
        + b2_ref[...]
    h2 = jnp.maximum(h2, 0.0)
    mu = jnp.mean(h2, axis=0, keepdims=True)
    var = jnp.mean((h2 - mu) ** 2, axis=0, keepdims=True)
    o_ref[...] = g_ref[...] * (h2 - mu) / jnp.sqrt(var + 1e-5) + bt_ref[...]


def _layer(first, inp, agg2, w1, b1, w2, b2, g, bt):
    return pl.pallas_call(
        functools.partial(_layer_body, first),
        out_shape=jax.ShapeDtypeStruct((N_NODES, DIM), jnp.float32),
    )(inp, agg2, w1, b1.reshape(1, DIM), w2, b2.reshape(1, DIM),
      g.reshape(1, DIM), bt.reshape(1, DIM))


def _head_body(h_ref, batch_ref, fc1w_ref, fc1b_ref, fc2w_ref, fc2b_ref,
               o_ref):
    h = h_ref[...]
    batch = batch_ref[...]
    gids = jax.lax.broadcasted_iota(jnp.int32, (NGRAPH, N_NODES), 0)
    onehot = jnp.where(gids == batch, 1.0, 0.0).astype(jnp.float32)
    pooled_sum = jnp.dot(onehot, h, preferred_element_type=jnp.float32)
    cnt = jnp.sum(onehot, axis=1, keepdims=True)
    pooled = pooled_sum / jnp.maximum(cnt, 1.0)
    z = jnp.maximum(
        jnp.dot(pooled, fc1w_ref[...], preferred_element_type=jnp.float32)
        + fc1b_ref[...], 0.0)
    z = jnp.dot(z, fc2w_ref[...], preferred_element_type=jnp.float32) \
        + fc2b_ref[...]
    m = jnp.max(z, axis=-1, keepdims=True)
    lse = jnp.log(jnp.sum(jnp.exp(z - m), axis=-1, keepdims=True)) + m
    o_ref[...] = z - lse


def _head(h, batch, fc1w, fc1b, fc2w, fc2b):
    return pl.pallas_call(
        _head_body,
        out_shape=jax.ShapeDtypeStruct((NGRAPH, NCLS), jnp.float32),
    )(h, batch.reshape(1, N_NODES), fc1w, fc1b.reshape(1, DIM), fc2w,
      fc2b.reshape(1, NCLS))


def _segsum_body(h_hbm, src_hbm, dst_hbm, out_hbm,
                 src_v, dst_v, rows_v, acc_sh, sem):
    c = lax.axis_index("c")
    s = lax.axis_index("s")
    wid = s * _NC + c

    z16 = jnp.zeros((16,), jnp.float32)

    def _zero_row(i, carry):
        rows_v[i, pl.ds(0, 16)] = z16
        rows_v[i, pl.ds(16, 16)] = z16
        return carry

    lax.fori_loop(0, _ZROWS, _zero_row, 0)
    pltpu.sync_copy(rows_v.at[pl.ds(0, _ZROWS)],
                    acc_sh.at[pl.ds(s * _ZROWS, _ZROWS)])
    plsc.subcore_barrier()

    for j in range(_NCHUNK):
        base = wid * _EPW + j * _CH
        pltpu.sync_copy(src_hbm.at[pl.ds(base, _CH)], src_v)
        pltpu.sync_copy(dst_hbm.at[pl.ds(base, _CH)], dst_v)
        pltpu.async_copy(h_hbm.at[src_v], rows_v, sem).wait()
        pltpu.sync_copy(rows_v, acc_sh.at[dst_v], add=True)

    plsc.subcore_barrier()
    pltpu.sync_copy(acc_sh.at[pl.ds(s * _ZROWS, _ZROWS)],
                    out_hbm.at[c, pl.ds(s * _ZROWS, _ZROWS)])


_segsum_call = pl.kernel(
    _segsum_body,
    out_type=jax.ShapeDtypeStruct((_NC, _NPAD, DIM), jnp.float32),
    mesh=plsc.VectorSubcoreMesh(core_axis_name="c", subcore_axis_name="s"),
    compiler_params=pltpu.CompilerParams(use_tc_tiling_on_sc=False),
    scratch_types=[
        pltpu.VMEM((_CH,), jnp.int32),
        pltpu.VMEM((_CH,), jnp.int32),
        pltpu.VMEM((_CH, DIM), jnp.float32),
        pltpu.VMEM_SHARED((_NPAD, DIM), jnp.float32),
        pltpu.SemaphoreType.DMA,
    ],
)


def _segsum(h, src, dst):
    return _segsum_call(h, src, dst)


def kernel(x, edge_index, batch,
           W1_1, b1_1, W2_1, b2_1, g_1, bt_1,
           W1_2, b1_2, W2_2, b2_2, g_2, bt_2,
           W1_3, b1_3, W2_3, b2_3, g_3, bt_3,
           W1_4, b1_4, W2_4, b2_4, g_4, bt_4,
           W1_5, b1_5, W2_5, b2_5, g_5, bt_5,
           fc1_W, fc1_b, fc2_W, fc2_b):
    src = edge_index[0]
    dst = edge_index[1]
    params = [
        (W1_1, b1_1, W2_1, b2_1, g_1, bt_1),
        (W1_2, b1_2, W2_2, b2_2, g_2, bt_2),
        (W1_3, b1_3, W2_3, b2_3, g_3, bt_3),
        (W1_4, b1_4, W2_4, b2_4, g_4, bt_4),
        (W1_5, b1_5, W2_5, b2_5, g_5, bt_5),
    ]
    h = _proj(x, W1_1)
    for l, (w1, b1, w2, b2, g, bt) in enumerate(params):
        agg2 = _segsum(h, src, dst)
        h = _layer(l == 0, h, agg2, w1, b1, w2, b2, g, bt)
    return _head(h, batch, fc1_W, fc1_b, fc2_W, fc2_b)


# speedup vs baseline: 20.1427x; 1.1421x over previous
import functools

import jax
import jax.numpy as jnp
from jax import lax
from jax.experimental import pallas as pl
from jax.experimental.pallas import tpu as pltpu
from jax.experimental.pallas import tpu_sc as plsc

N_NODES = 10000
DIM = 32
NGRAPH = 64
NCLS = 10
N_EDGES = 320000

_NC = 2
_NS = 16
_NW = _NC * _NS
_EPW = N_EDGES // _NW
_CH = 1000
_NCHUNK = _EPW // _CH
_NPAD = 10240
_ZROWS = _NPAD // _NS


def _proj_body(x_ref, w_ref, o_ref):
    o_ref[...] = jnp.dot(x_ref[...], w_ref[...],
                         preferred_element_type=jnp.float32)


def _proj(x, w):
    return pl.pallas_call(
        _proj_body,
        out_shape=jax.ShapeDtypeStruct((x.shape[0], w.shape[1]), jnp.float32),
    )(x, w)


def _layer_body(first, inp_ref, agg_ref, w1_ref, b1_ref, w2_ref, b2_ref,
                g_ref, bt_ref, o_ref):
    u = inp_ref[...] + agg_ref[0, :N_NODES] + agg_ref[1, :N_NODES]
    if first:
        h2 = jnp.maximum(u + b1_ref[...], 0.0)
    else:
        h2 = jnp.maximum(
            jnp.dot(u, w1_ref[...], preferred_element_type=jnp.float32)
            + b1_ref[...], 0.0)
    h2 = jnp.dot(h2, w2_ref[...], preferred_element_type=jnp.float32) \
        + b2_ref[...]
    h2 = jnp.maximum(h2, 0.0)
    mu = jnp.mean(h2, axis=0, keepdims=True)
    var = jnp.mean((h2 - mu) ** 2, axis=0, keepdims=True)
    o_ref[...] = g_ref[...] * (h2 - mu) / jnp.sqrt(var + 1e-5) + bt_ref[...]


def _layer(first, inp, agg2, w1, b1, w2, b2, g, bt):
    return pl.pallas_call(
        functools.partial(_layer_body, first),
        out_shape=jax.ShapeDtypeStruct((N_NODES, DIM), jnp.float32),
    )(inp, agg2, w1, b1.reshape(1, DIM), w2, b2.reshape(1, DIM),
      g.reshape(1, DIM), bt.reshape(1, DIM))


def _head_body(h_ref, batch_ref, fc1w_ref, fc1b_ref, fc2w_ref, fc2b_ref,
               o_ref):
    h = h_ref[...]
    batch = batch_ref[...]
    gids = jax.lax.broadcasted_iota(jnp.int32, (NGRAPH, N_NODES), 0)
    onehot = jnp.where(gids == batch, 1.0, 0.0).astype(jnp.float32)
    pooled_sum = jnp.dot(onehot, h, preferred_element_type=jnp.float32)
    cnt = jnp.sum(onehot, axis=1, keepdims=True)
    pooled = pooled_sum / jnp.maximum(cnt, 1.0)
    z = jnp.maximum(
        jnp.dot(pooled, fc1w_ref[...], preferred_element_type=jnp.float32)
        + fc1b_ref[...], 0.0)
    z = jnp.dot(z, fc2w_ref[...], preferred_element_type=jnp.float32) \
        + fc2b_ref[...]
    m = jnp.max(z, axis=-1, keepdims=True)
    lse = jnp.log(jnp.sum(jnp.exp(z - m), axis=-1, keepdims=True)) + m
    o_ref[...] = z - lse


def _head(h, batch, fc1w, fc1b, fc2w, fc2b):
    return pl.pallas_call(
        _head_body,
        out_shape=jax.ShapeDtypeStruct((NGRAPH, NCLS), jnp.float32),
    )(h, batch.reshape(1, N_NODES), fc1w, fc1b.reshape(1, DIM), fc2w,
      fc2b.reshape(1, NCLS))


def _segsum_body(h_hbm, src_hbm, dst_hbm, out_hbm,
                 src_v, dst_v, rows_v, zero_v, acc_sh,
                 gsem, ssem):
    c = lax.axis_index("c")
    s = lax.axis_index("s")
    wid = s * _NC + c

    z16 = jnp.zeros((16,), jnp.float32)

    def _zero_row(i, carry):
        zero_v[i, pl.ds(0, 16)] = z16
        zero_v[i, pl.ds(16, 16)] = z16
        return carry

    lax.fori_loop(0, _ZROWS, _zero_row, 0)
    pltpu.sync_copy(zero_v, acc_sh.at[pl.ds(s * _ZROWS, _ZROWS)])
    plsc.subcore_barrier()

    def _load_idx(j):
        base = wid * _EPW + j * _CH
        b = j % 2
        pltpu.sync_copy(src_hbm.at[pl.ds(base, _CH)], src_v.at[b])
        pltpu.sync_copy(dst_hbm.at[pl.ds(base, _CH)], dst_v.at[b])

    _load_idx(0)
    gather0 = pltpu.async_copy(h_hbm.at[src_v.at[0]], rows_v.at[0], gsem)
    gathers = [gather0]
    scatters = []
    for j in range(_NCHUNK):
        b = j % 2
        if j + 1 < _NCHUNK:
            if j - 1 >= 0:
                scatters[j - 1].wait()
            _load_idx(j + 1)
            gathers.append(pltpu.async_copy(
                h_hbm.at[src_v.at[1 - b]], rows_v.at[1 - b], gsem))
        gathers[j].wait()
        scatters.append(pltpu.async_copy(
            rows_v.at[b], acc_sh.at[dst_v.at[b]], ssem, add=True))
    scatters[_NCHUNK - 2].wait()
    scatters[_NCHUNK - 1].wait()

    plsc.subcore_barrier()
    pltpu.sync_copy(acc_sh.at[pl.ds(s * _ZROWS, _ZROWS)],
                    out_hbm.at[c, pl.ds(s * _ZROWS, _ZROWS)])


_segsum_call = pl.kernel(
    _segsum_body,
    out_type=jax.ShapeDtypeStruct((_NC, _NPAD, DIM), jnp.float32),
    mesh=plsc.VectorSubcoreMesh(core_axis_name="c", subcore_axis_name="s"),
    compiler_params=pltpu.CompilerParams(use_tc_tiling_on_sc=False),
    scratch_types=[
        pltpu.VMEM((2, _CH), jnp.int32),
        pltpu.VMEM((2, _CH), jnp.int32),
        pltpu.VMEM((2, _CH, DIM), jnp.float32),
        pltpu.VMEM((_ZROWS, DIM), jnp.float32),
        pltpu.VMEM_SHARED((_NPAD, DIM), jnp.float32),
        pltpu.SemaphoreType.DMA,
        pltpu.SemaphoreType.DMA,
    ],
)


def _segsum(h, src, dst):
    return _segsum_call(h, src, dst)


def kernel(x, edge_index, batch,
           W1_1, b1_1, W2_1, b2_1, g_1, bt_1,
           W1_2, b1_2, W2_2, b2_2, g_2, bt_2,
           W1_3, b1_3, W2_3, b2_3, g_3, bt_3,
           W1_4, b1_4, W2_4, b2_4, g_4, bt_4,
           W1_5, b1_5, W2_5, b2_5, g_5, bt_5,
           fc1_W, fc1_b, fc2_W, fc2_b):
    src = edge_index[0]
    dst = edge_index[1]
    params = [
        (W1_1, b1_1, W2_1, b2_1, g_1, bt_1),
        (W1_2, b1_2, W2_2, b2_2, g_2, bt_2),
        (W1_3, b1_3, W2_3, b2_3, g_3, bt_3),
        (W1_4, b1_4, W2_4, b2_4, g_4, bt_4),
        (W1_5, b1_5, W2_5, b2_5, g_5, bt_5),
    ]
    h = _proj(x, W1_1)
    for l, (w1, b1, w2, b2, g, bt) in enumerate(params):
        agg2 = _segsum(h, src, dst)
        h = _layer(l == 0, h, agg2, w1, b1, w2, b2, g, bt)
    return _head(h, batch, fc1_W, fc1_b, fc2_W, fc2_b)


# speedup vs baseline: 20.4066x; 1.0131x over previous
import functools

import jax
import jax.numpy as jnp
from jax import lax
from jax.experimental import pallas as pl
from jax.experimental.pallas import tpu as pltpu
from jax.experimental.pallas import tpu_sc as plsc

N_NODES = 10000
DIM = 32
NGRAPH = 64
NCLS = 10
N_EDGES = 320000

_NC = 2
_NS = 16
_NW = _NC * _NS
_EPW = N_EDGES // _NW
_CH = 1000
_NCHUNK = _EPW // _CH
_NPAD = 10240
_ZROWS = _NPAD // _NS


def _proj_body(x_ref, w_ref, o_ref):
    o_ref[...] = jnp.dot(x_ref[...], w_ref[...],
                         preferred_element_type=jnp.float32)


def _proj(x, w):
    return pl.pallas_call(
        _proj_body,
        out_shape=jax.ShapeDtypeStruct((x.shape[0], w.shape[1]), jnp.float32),
    )(x, w)


def _layer_body(first, inp_ref, agg_ref, w1_ref, b1_ref, w2_ref, b2_ref,
                g_ref, bt_ref, o_ref):
    u = inp_ref[...] + agg_ref[0, :N_NODES] + agg_ref[1, :N_NODES]
    if first:
        h2 = jnp.maximum(u + b1_ref[...], 0.0)
    else:
        h2 = jnp.maximum(
            jnp.dot(u, w1_ref[...], preferred_element_type=jnp.float32)
            + b1_ref[...], 0.0)
    h2 = jnp.dot(h2, w2_ref[...], preferred_element_type=jnp.float32) \
        + b2_ref[...]
    h2 = jnp.maximum(h2, 0.0)
    mu = jnp.mean(h2, axis=0, keepdims=True)
    var = jnp.mean((h2 - mu) ** 2, axis=0, keepdims=True)
    o_ref[...] = g_ref[...] * (h2 - mu) / jnp.sqrt(var + 1e-5) + bt_ref[...]


def _layer(first, inp, agg2, w1, b1, w2, b2, g, bt):
    return pl.pallas_call(
        functools.partial(_layer_body, first),
        out_shape=jax.ShapeDtypeStruct((N_NODES, DIM), jnp.float32),
    )(inp, agg2, w1, b1.reshape(1, DIM), w2, b2.reshape(1, DIM),
      g.reshape(1, DIM), bt.reshape(1, DIM))


def _head_body(h_ref, batch_ref, fc1w_ref, fc1b_ref, fc2w_ref, fc2b_ref,
               o_ref):
    h = h_ref[...]
    batch = batch_ref[...]
    gids = jax.lax.broadcasted_iota(jnp.int32, (NGRAPH, N_NODES), 0)
    onehot = jnp.where(gids == batch, 1.0, 0.0).astype(jnp.float32)
    pooled_sum = jnp.dot(onehot, h, preferred_element_type=jnp.float32)
    cnt = jnp.sum(onehot, axis=1, keepdims=True)
    pooled = pooled_sum / jnp.maximum(cnt, 1.0)
    z = jnp.maximum(
        jnp.dot(pooled, fc1w_ref[...], preferred_element_type=jnp.float32)
        + fc1b_ref[...], 0.0)
    z = jnp.dot(z, fc2w_ref[...], preferred_element_type=jnp.float32) \
        + fc2b_ref[...]
    m = jnp.max(z, axis=-1, keepdims=True)
    lse = jnp.log(jnp.sum(jnp.exp(z - m), axis=-1, keepdims=True)) + m
    o_ref[...] = z - lse


def _head(h, batch, fc1w, fc1b, fc2w, fc2b):
    return pl.pallas_call(
        _head_body,
        out_shape=jax.ShapeDtypeStruct((NGRAPH, NCLS), jnp.float32),
    )(h, batch.reshape(1, N_NODES), fc1w, fc1b.reshape(1, DIM), fc2w,
      fc2b.reshape(1, NCLS))


def _segsum_body(h_hbm, src_hbm, dst_hbm, out_hbm,
                 src_v, dst_v, rows_v, zero_v, acc_sh,
                 gsem, ssem):
    c = lax.axis_index("c")
    s = lax.axis_index("s")
    wid = s * _NC + c

    def _load_idx(j):
        base = wid * _EPW + j * _CH
        b = j % 2
        pltpu.sync_copy(src_hbm.at[pl.ds(base, _CH)], src_v.at[b])
        pltpu.sync_copy(dst_hbm.at[pl.ds(base, _CH)], dst_v.at[b])

    _load_idx(0)
    gather0 = pltpu.async_copy(h_hbm.at[src_v.at[0]], rows_v.at[0], gsem)

    z16 = jnp.zeros((16,), jnp.float32)

    def _zero_row(i, carry):
        zero_v[i, pl.ds(0, 16)] = z16
        zero_v[i, pl.ds(16, 16)] = z16
        return carry

    lax.fori_loop(0, _ZROWS, _zero_row, 0)
    pltpu.sync_copy(zero_v, acc_sh.at[pl.ds(s * _ZROWS, _ZROWS)])
    plsc.subcore_barrier()

    gathers = [gather0]
    scatters = []
    for j in range(_NCHUNK):
        b = j % 2
        if j + 1 < _NCHUNK:
            if j - 1 >= 0:
                scatters[j - 1].wait()
            _load_idx(j + 1)
            gathers.append(pltpu.async_copy(
                h_hbm.at[src_v.at[1 - b]], rows_v.at[1 - b], gsem))
        gathers[j].wait()
        scatters.append(pltpu.async_copy(
            rows_v.at[b], acc_sh.at[dst_v.at[b]], ssem, add=True))
    scatters[_NCHUNK - 2].wait()
    scatters[_NCHUNK - 1].wait()

    plsc.subcore_barrier()
    pltpu.sync_copy(acc_sh.at[pl.ds(s * _ZROWS, _ZROWS)],
                    out_hbm.at[c, pl.ds(s * _ZROWS, _ZROWS)])


_segsum_call = pl.kernel(
    _segsum_body,
    out_type=jax.ShapeDtypeStruct((_NC, _NPAD, DIM), jnp.float32),
    mesh=plsc.VectorSubcoreMesh(core_axis_name="c", subcore_axis_name="s"),
    compiler_params=pltpu.CompilerParams(use_tc_tiling_on_sc=False),
    scratch_types=[
        pltpu.VMEM((2, _CH), jnp.int32),
        pltpu.VMEM((2, _CH), jnp.int32),
        pltpu.VMEM((2, _CH, DIM), jnp.float32),
        pltpu.VMEM((_ZROWS, DIM), jnp.float32),
        pltpu.VMEM_SHARED((_NPAD, DIM), jnp.float32),
        pltpu.SemaphoreType.DMA,
        pltpu.SemaphoreType.DMA,
    ],
)


def _segsum(h, src, dst):
    return _segsum_call(h, src, dst)


def kernel(x, edge_index, batch,
           W1_1, b1_1, W2_1, b2_1, g_1, bt_1,
           W1_2, b1_2, W2_2, b2_2, g_2, bt_2,
           W1_3, b1_3, W2_3, b2_3, g_3, bt_3,
           W1_4, b1_4, W2_4, b2_4, g_4, bt_4,
           W1_5, b1_5, W2_5, b2_5, g_5, bt_5,
           fc1_W, fc1_b, fc2_W, fc2_b):
    src = edge_index[0]
    dst = edge_index[1]
    params = [
        (W1_1, b1_1, W2_1, b2_1, g_1, bt_1),
        (W1_2, b1_2, W2_2, b2_2, g_2, bt_2),
        (W1_3, b1_3, W2_3, b2_3, g_3, bt_3),
        (W1_4, b1_4, W2_4, b2_4, g_4, bt_4),
        (W1_5, b1_5, W2_5, b2_5, g_5, bt_5),
    ]
    h = _proj(x, W1_1)
    for l, (w1, b1, w2, b2, g, bt) in enumerate(params):
        agg2 = _segsum(h, src, dst)
        h = _layer(l == 0, h, agg2, w1, b1, w2, b2, g, bt)
    return _head(h, batch, fc1_W, fc1_b, fc2_W, fc2_b)
